# baseline (speedup 1.0000x reference)
import functools

import jax
import jax.numpy as jnp
from jax import lax
from jax.experimental import pallas as pl
from jax.experimental.pallas import tpu as pltpu
from jax.experimental.pallas import tpu_sc as plsc

SEQ = 12
NL = 2
N = 10000
D = 128
D2 = 2 * D
NQ = 4
DQ = D // NQ
NPAD = 10240
CHUNK = 512
NBUF = 2
LAG = 1
NC = 2
NS = 16
RPT = NPAD // NS
CPT = 40
EPAD = NS * CPT * CHUNK
BN = 1280
NBLK = NPAD // BN
LANES = 128


@functools.lru_cache(maxsize=None)
def _sc_agg(n_tab: int):
    mesh = plsc.VectorSubcoreMesh(core_axis_name="c", subcore_axis_name="s")
    out_type = [jax.ShapeDtypeStruct((n_tab, NQ, NPAD, DQ), jnp.float32)]
    scratch_types = [
        pltpu.VMEM((CPT, CHUNK), jnp.int32),
        pltpu.VMEM((CPT, CHUNK), jnp.int32),
        pltpu.VMEM((NBUF, CHUNK, DQ), jnp.float32),
        pltpu.VMEM((LANES, DQ), jnp.float32),
        pltpu.VMEM_SHARED((NPAD, DQ), jnp.float32),
        pltpu.VMEM_SHARED((NPAD, DQ), jnp.float32),
    ] + [pltpu.SemaphoreType.DMA for _ in range(2 * NBUF)]

    def body(*refs):
        it = iter(refs)
        tabs = [next(it) for _ in range(n_tab)]
        src3 = next(it)
        dst3 = next(it)
        zrow_h = next(it)
        out = next(it)
        srcv = next(it)
        dstv = next(it)
        ring = next(it)
        zbuf = next(it)
        tstage = next(it)
        acc = next(it)
        gsems = [next(it) for _ in range(NBUF)]
        ssems = [next(it) for _ in range(NBUF)]

        cid = lax.axis_index("c")
        sid = lax.axis_index("s")
        r0 = sid * RPT

        pltpu.sync_copy(src3.at[sid], srcv)
        pltpu.sync_copy(dst3.at[sid], dstv)
        pltpu.sync_copy(zrow_h, zbuf)

        for t in range(n_tab):
            for qq in range(NQ // NC):
                qg = cid * (NQ // NC) + qq
                pltpu.sync_copy(tabs[t].at[pl.ds(qg * NPAD + r0, RPT)],
                                tstage.at[pl.ds(r0, RPT)])
                for k in range(RPT // LANES):
                    pltpu.sync_copy(zbuf,
                                    acc.at[pl.ds(r0 + k * LANES, LANES)])
                plsc.subcore_barrier()
                for b in range(NBUF):
                    pltpu.async_copy(tstage.at[srcv.at[b]], ring.at[b],
                                     gsems[b])

                def group(g, carry):
                    for b in range(NBUF):
                        c = g * NBUF + b
                        pltpu.make_async_copy(
                            tstage.at[srcv.at[c]], ring.at[b],
                            gsems[b]).wait()
                        pltpu.async_copy(ring.at[b], acc.at[dstv.at[c]],
                                         ssems[b], add=True)
                        br = (b - LAG) % NBUF
                        cr = c - LAG
                        nxt = cr + NBUF

                        @pl.when((cr >= 0) & (nxt < CPT))
                        def _():
                            pltpu.make_async_copy(
                                ring.at[br], acc.at[dstv.at[cr]],
                                ssems[br]).wait()
                            pltpu.async_copy(
                                tstage.at[srcv.at[nxt]], ring.at[br],
                                gsems[br])
                    return carry

                lax.fori_loop(0, CPT // NBUF, group, 0)
                for b in range(NBUF):
                    c = CPT - NBUF + b
                    pltpu.make_async_copy(
                        ring.at[b], acc.at[dstv.at[c]], ssems[b]).wait()
                plsc.subcore_barrier()
                pltpu.sync_copy(acc.at[pl.ds(r0, RPT)],
                                out.at[t, qg, pl.ds(r0, RPT)])

    return pl.kernel(body, out_type=out_type, mesh=mesh,
                     scratch_types=scratch_types,
                     compiler_params=pltpu.CompilerParams(
                         use_tc_tiling_on_sc=False))


@functools.lru_cache(maxsize=None)
def _gate(has_x: bool, has_h: bool, emit_out: bool):
    q_spec = pl.BlockSpec((NQ, BN, DQ), lambda i: (0, i, 0))
    in_specs = []
    if has_x:
        in_specs.append(q_spec)
    if has_h:
        in_specs.append(q_spec)
        in_specs.append(q_spec)
    in_specs.append(pl.BlockSpec((BN, 1), lambda i: (i, 0)))
    if has_x:
        in_specs.append(pl.BlockSpec((D, D2), lambda i: (0, 0)))
    if has_h:
        in_specs.append(pl.BlockSpec((D, D2), lambda i: (0, 0)))
    in_specs.append(pl.BlockSpec((1, D2), lambda i: (0, 0)))
    if emit_out:
        in_specs.append(pl.BlockSpec((D, D), lambda i: (0, 0)))
        in_specs.append(pl.BlockSpec((1, D), lambda i: (0, 0)))
    out_specs = [pl.BlockSpec((NQ, BN, DQ), lambda i: (0, i, 0))]
    out_shape = [jax.ShapeDtypeStruct((NQ, NPAD, DQ), jnp.float32)]
    if emit_out:
        out_specs.append(pl.BlockSpec((BN, D), lambda i: (i, 0)))
        out_shape.append(jax.ShapeDtypeStruct((NPAD, D), jnp.float32))

    def body(*refs):
        it = iter(refs)
        axp = next(it) if has_x else None
        if has_h:
            ahp = next(it)
            h4 = next(it)
        invd = next(it)
        Wx = next(it) if has_x else None
        Wh = next(it) if has_h else None
        bias = next(it)
        if emit_out:
            ow = next(it)
            ob = next(it)
        hn_ref = next(it)
        out_ref = next(it) if emit_out else None

        def full(q):
            return jnp.concatenate([q[k] for k in range(NQ)], axis=1)

        iv = invd[...]
        G = bias[...]
        if has_x:
            G = G + jnp.dot(full(axp) * iv, Wx[...],
                            preferred_element_type=jnp.float32)
        if has_h:
            G = G + jnp.dot(full(ahp) * iv, Wh[...],
                            preferred_element_type=jnp.float32)
        u = jax.nn.sigmoid(G[:, :D])
        c = jnp.tanh(G[:, D:])
        hn = u * full(h4) + (1.0 - u) * c if has_h else (1.0 - u) * c
        rows = (pl.program_id(0) * BN
                + lax.broadcasted_iota(jnp.int32, (BN, 1), 0))
        hn = jnp.where(rows < N, hn, 0.0)
        for k in range(NQ):
            hn_ref[k] = hn[:, k * DQ:(k + 1) * DQ]
        if emit_out:
            out_ref[...] = (jnp.dot(hn, ow[...],
                                    preferred_element_type=jnp.float32)
                            + ob[...])

    return pl.pallas_call(body, grid=(NBLK,), in_specs=in_specs,
                          out_specs=out_specs, out_shape=out_shape)


@functools.lru_cache(maxsize=None)
def _gate2(b_has_x: bool, emit_out: bool):
    q_spec = pl.BlockSpec((NQ, BN, DQ), lambda i: (0, i, 0))
    w_spec = pl.BlockSpec((D, D2), lambda i: (0, 0))
    b_spec = pl.BlockSpec((1, D2), lambda i: (0, 0))
    in_specs = [q_spec, q_spec, q_spec]
    if b_has_x:
        in_specs.append(q_spec)
    in_specs.append(q_spec)
    in_specs.append(pl.BlockSpec((BN, 1), lambda i: (i, 0)))
    in_specs += [w_spec, w_spec, b_spec]
    if b_has_x:
        in_specs.append(w_spec)
    in_specs += [w_spec, b_spec]
    if emit_out:
        in_specs.append(pl.BlockSpec((D, D), lambda i: (0, 0)))
        in_specs.append(pl.BlockSpec((1, D), lambda i: (0, 0)))
    out_specs = [pl.BlockSpec((NQ, BN, DQ), lambda i: (0, i, 0))] * 2
    out_shape = [jax.ShapeDtypeStruct((NQ, NPAD, DQ), jnp.float32)] * 2
    if emit_out:
        out_specs = out_specs + [pl.BlockSpec((BN, D), lambda i: (i, 0))]
        out_shape = out_shape + [jax.ShapeDtypeStruct((NPAD, D),
                                                      jnp.float32)]

    def body(*refs):
        it = iter(refs)
        pa = next(it)
        pb = next(it)
        h1 = next(it)
        iaB = next(it) if b_has_x else None
        h0 = next(it)
        invd = next(it)
        WxA = next(it)
        WhA = next(it)
        bA = next(it)
        WxB = next(it) if b_has_x else None
        WhB = next(it)
        bB = next(it)
        if emit_out:
            ow = next(it)
            ob = next(it)
        h1n_ref = next(it)
        h0n_ref = next(it)
        out_ref = next(it) if emit_out else None

        def full(q):
            return jnp.concatenate([q[k] for k in range(NQ)], axis=1)

        iv = invd[...]
        fpa = full(pa) * iv
        rows = (pl.program_id(0) * BN
                + lax.broadcasted_iota(jnp.int32, (BN, 1), 0))
        msk = rows < N

        GA = (bA[...]
              + jnp.dot(fpa, WxA[...], preferred_element_type=jnp.float32)
              + jnp.dot(full(pb) * iv, WhA[...],
                        preferred_element_type=jnp.float32))
        uA = jax.nn.sigmoid(GA[:, :D])
        cA = jnp.tanh(GA[:, D:])
        h1n = jnp.where(msk, uA * full(h1) + (1.0 - uA) * cA, 0.0)

        GB = bB[...] + jnp.dot(fpa, WhB[...],
                               preferred_element_type=jnp.float32)
        if b_has_x:
            GB = GB + jnp.dot(full(iaB) * iv, WxB[...],
                              preferred_element_type=jnp.float32)
        uB = jax.nn.sigmoid(GB[:, :D])
        cB = jnp.tanh(GB[:, D:])
        h0n = jnp.where(msk, uB * full(h0) + (1.0 - uB) * cB, 0.0)

        for k in range(NQ):
            h1n_ref[k] = h1n[:, k * DQ:(k + 1) * DQ]
            h0n_ref[k] = h0n[:, k * DQ:(k + 1) * DQ]
        if emit_out:
            out_ref[...] = (jnp.dot(h1n, ow[...],
                                    preferred_element_type=jnp.float32)
                            + ob[...])

    return pl.pallas_call(body, grid=(NBLK,), in_specs=in_specs,
                          out_specs=out_specs, out_shape=out_shape)


def kernel(inputs, teacher_states, enc_W, enc_b, dec_W, dec_b, out_W, out_b,
           edge_index, batch_cnt):
    f32 = jnp.float32
    src = edge_index[0]
    dst = edge_index[1]
    E = src.shape[0]
    pad_e = EPAD - E
    src3 = jnp.concatenate(
        [src, jnp.full((pad_e,), N, jnp.int32)]).reshape(NS, CPT, CHUNK)
    dst3 = jnp.concatenate(
        [dst, jnp.zeros((pad_e,), jnp.int32)]).reshape(NS, CPT, CHUNK)
    zrow = jnp.zeros((LANES, DQ), f32)

    xs = jnp.zeros((SEQ, NQ, NPAD, DQ), f32).at[:, :, :N, :].set(
        jnp.moveaxis(inputs.reshape(SEQ, N, NQ, DQ), 2, 1))
    xtabs = [xs[i].reshape(NQ * NPAD, DQ) for i in range(SEQ)]
    ones_tab = jnp.zeros((NQ, NPAD, DQ), f32).at[:, :N, :].set(1.0)
    ones_tab = ones_tab.reshape(NQ * NPAD, DQ)

    agg1 = _sc_agg(1)
    agg2 = _sc_agg(2)
    gate_x = _gate(True, False, False)
    gate_xh = _gate(True, True, False)
    gate_h = _gate(False, True, False)
    gate_xh_o = _gate(True, True, True)

    def agg(*tabs):
        return _sc_agg(len(tabs))(*tabs, src3, dst3, zrow)[0]

    p = agg(ones_tab, *xtabs[:6])
    degp = p[0]
    deg = degp[0, :, 0]
    invd = (1.0 / jnp.maximum(deg, 1.0))[:, None]
    IA = list(p[1:7]) + list(agg(*xtabs[6:])) + [None] * 0

    def mk(Ws, bs):
        Wx = jnp.concatenate([Ws[2], Ws[4]], axis=1)
        Wh = jnp.concatenate([Ws[3], Ws[5]], axis=1)
        b = jnp.concatenate([bs[2] + bs[3], bs[4] + bs[5]])[None, :]
        return Wx, Wh, b

    encp = [mk(enc_W[j], enc_b[j]) for j in range(NL)]
    decp = [mk(dec_W[j], dec_b[j]) for j in range(NL)]
    ob = out_b[None, :]

    def flat(h4):
        return h4.reshape(NQ * NPAD, DQ)

    pair_enc = _gate2(True, False)
    pair_ed = _gate2(False, False)
    pair_dec = _gate2(False, True)

    h0 = gate_x(IA[0], invd, encp[0][0], encp[0][2])[0]
    ap_h0 = agg(flat(h0))[0]
    h1 = gate_x(ap_h0, invd, encp[1][0], encp[1][2])[0]
    h0 = gate_xh(IA[1], ap_h0, h0, invd,
                 encp[0][0], encp[0][1], encp[0][2])[0]
    for i in range(1, SEQ):
        pa, pb = agg(flat(h0), flat(h1))[:2]
        if i < SEQ - 1:
            h1, h0 = pair_enc(pa, pb, h1, IA[i + 1], h0, invd,
                              encp[1][0], encp[1][1], encp[1][2],
                              encp[0][0], encp[0][1], encp[0][2])
        else:
            h1, h0 = pair_ed(pa, pb, h1, h0, invd,
                             encp[1][0], encp[1][1], encp[1][2],
                             decp[0][1], decp[0][2])

    outs = []
    for i in range(SEQ):
        pa, pb = agg(flat(h0), flat(h1))[:2]
        if i < SEQ - 1:
            h1, h0, o = pair_dec(pa, pb, h1, h0, invd,
                                 decp[1][0], decp[1][1], decp[1][2],
                                 decp[0][1], decp[0][2], out_W, ob)
        else:
            h1, o = gate_xh_o(pa, pb, h1, invd,
                              decp[1][0], decp[1][1], decp[1][2], out_W, ob)
        outs.append(o[:N])
    return jnp.stack(outs)


# speedup vs baseline: 1.1376x; 1.1376x over previous
import functools

import jax
import jax.numpy as jnp
from jax import lax
from jax.experimental import pallas as pl
from jax.experimental.pallas import tpu as pltpu
from jax.experimental.pallas import tpu_sc as plsc

SEQ = 12
NL = 2
N = 10000
D = 128
D2 = 2 * D
NQ = 4
DQ = D // NQ
NPAD = 10240
CHUNK = 512
NBUF = 2
LAG = 1
NC = 2
NS = 16
RPT = NPAD // NS
CPT = 40
EPAD = NS * CPT * CHUNK
BN = 1280
NBLK = NPAD // BN
LANES = 128


@functools.lru_cache(maxsize=None)
def _sc_agg(n_tab: int):
    mesh = plsc.VectorSubcoreMesh(core_axis_name="c", subcore_axis_name="s")
    out_type = [jax.ShapeDtypeStruct((n_tab, NQ, NPAD, DQ), jnp.float32)]
    scratch_types = [
        pltpu.VMEM((CPT, CHUNK), jnp.int32),
        pltpu.VMEM((CPT, CHUNK), jnp.int32),
        pltpu.VMEM((NBUF, CHUNK, DQ), jnp.float32),
        pltpu.VMEM((LANES, DQ), jnp.float32),
        pltpu.VMEM_SHARED((NPAD, DQ), jnp.float32),
        pltpu.VMEM_SHARED((NPAD, DQ), jnp.float32),
    ] + [pltpu.SemaphoreType.DMA for _ in range(2 * NBUF)]

    def body(*refs):
        it = iter(refs)
        tabs = [next(it) for _ in range(n_tab)]
        src3 = next(it)
        dst3 = next(it)
        zrow_h = next(it)
        out = next(it)
        srcv = next(it)
        dstv = next(it)
        ring = next(it)
        zbuf = next(it)
        tstage = next(it)
        acc = next(it)
        gsems = [next(it) for _ in range(NBUF)]
        ssems = [next(it) for _ in range(NBUF)]

        cid = lax.axis_index("c")
        sid = lax.axis_index("s")
        r0 = sid * RPT

        pltpu.sync_copy(src3.at[sid], srcv)
        pltpu.sync_copy(dst3.at[sid], dstv)
        pltpu.sync_copy(zrow_h, zbuf)

        for t in range(n_tab):
            for qq in range(NQ // NC):
                qg = cid * (NQ // NC) + qq
                pltpu.sync_copy(tabs[t].at[pl.ds(qg * NPAD + r0, RPT)],
                                tstage.at[pl.ds(r0, RPT)])
                for k in range(RPT // LANES):
                    pltpu.sync_copy(zbuf,
                                    acc.at[pl.ds(r0 + k * LANES, LANES)])
                plsc.subcore_barrier()
                for b in range(NBUF):
                    pltpu.async_copy(tstage.at[srcv.at[b]], ring.at[b],
                                     gsems[b])

                def group(g, carry):
                    for b in range(NBUF):
                        c = g * NBUF + b
                        pltpu.make_async_copy(
                            tstage.at[srcv.at[c]], ring.at[b],
                            gsems[b]).wait()
                        pltpu.async_copy(ring.at[b], acc.at[dstv.at[c]],
                                         ssems[b], add=True)
                        br = (b - LAG) % NBUF
                        cr = c - LAG
                        nxt = cr + NBUF

                        @pl.when((cr >= 0) & (nxt < CPT))
                        def _():
                            pltpu.make_async_copy(
                                ring.at[br], acc.at[dstv.at[cr]],
                                ssems[br]).wait()
                            pltpu.async_copy(
                                tstage.at[srcv.at[nxt]], ring.at[br],
                                gsems[br])
                    return carry

                lax.fori_loop(0, CPT // NBUF, group, 0)
                for b in range(NBUF):
                    c = CPT - NBUF + b
                    pltpu.make_async_copy(
                        ring.at[b], acc.at[dstv.at[c]], ssems[b]).wait()
                plsc.subcore_barrier()
                pltpu.sync_copy(acc.at[pl.ds(r0, RPT)],
                                out.at[t, qg, pl.ds(r0, RPT)])

    return pl.kernel(body, out_type=out_type, mesh=mesh,
                     scratch_types=scratch_types,
                     compiler_params=pltpu.CompilerParams(
                         use_tc_tiling_on_sc=False))


@functools.lru_cache(maxsize=None)
def _gate(has_x: bool, has_h: bool, emit_out: bool):
    q_spec = pl.BlockSpec((NQ, BN, DQ), lambda i: (0, i, 0))
    in_specs = []
    if has_x:
        in_specs.append(q_spec)
    if has_h:
        in_specs.append(q_spec)
        in_specs.append(q_spec)
    in_specs.append(pl.BlockSpec((BN, 1), lambda i: (i, 0)))
    if has_x:
        in_specs.append(pl.BlockSpec((D, D2), lambda i: (0, 0)))
    if has_h:
        in_specs.append(pl.BlockSpec((D, D2), lambda i: (0, 0)))
    in_specs.append(pl.BlockSpec((1, D2), lambda i: (0, 0)))
    if emit_out:
        in_specs.append(pl.BlockSpec((D, D), lambda i: (0, 0)))
        in_specs.append(pl.BlockSpec((1, D), lambda i: (0, 0)))
    out_specs = [pl.BlockSpec((NQ, BN, DQ), lambda i: (0, i, 0))]
    out_shape = [jax.ShapeDtypeStruct((NQ, NPAD, DQ), jnp.float32)]
    if emit_out:
        out_specs.append(pl.BlockSpec((BN, D), lambda i: (i, 0)))
        out_shape.append(jax.ShapeDtypeStruct((NPAD, D), jnp.float32))

    def body(*refs):
        it = iter(refs)
        axp = next(it) if has_x else None
        if has_h:
            ahp = next(it)
            h4 = next(it)
        invd = next(it)
        Wx = next(it) if has_x else None
        Wh = next(it) if has_h else None
        bias = next(it)
        if emit_out:
            ow = next(it)
            ob = next(it)
        hn_ref = next(it)
        out_ref = next(it) if emit_out else None

        def full(q):
            return jnp.concatenate([q[k] for k in range(NQ)], axis=1)

        iv = invd[...]
        G = bias[...]
        if has_x:
            G = G + jnp.dot(full(axp) * iv, Wx[...],
                            preferred_element_type=jnp.float32)
        if has_h:
            G = G + jnp.dot(full(ahp) * iv, Wh[...],
                            preferred_element_type=jnp.float32)
        u = jax.nn.sigmoid(G[:, :D])
        c = jnp.tanh(G[:, D:])
        hn = u * full(h4) + (1.0 - u) * c if has_h else (1.0 - u) * c
        rows = (pl.program_id(0) * BN
                + lax.broadcasted_iota(jnp.int32, (BN, 1), 0))
        hn = jnp.where(rows < N, hn, 0.0)
        for k in range(NQ):
            hn_ref[k] = hn[:, k * DQ:(k + 1) * DQ]
        if emit_out:
            out_ref[...] = (jnp.dot(hn, ow[...],
                                    preferred_element_type=jnp.float32)
                            + ob[...])

    return pl.pallas_call(body, grid=(NBLK,), in_specs=in_specs,
                          out_specs=out_specs, out_shape=out_shape)


@functools.lru_cache(maxsize=None)
def _gate2(b_has_x: bool, emit_out: bool):
    q_spec = pl.BlockSpec((NQ, BN, DQ), lambda i: (0, i, 0))
    w_spec = pl.BlockSpec((D, D2), lambda i: (0, 0))
    b_spec = pl.BlockSpec((1, D2), lambda i: (0, 0))
    in_specs = [q_spec, q_spec, q_spec]
    if b_has_x:
        in_specs.append(q_spec)
    in_specs.append(q_spec)
    in_specs.append(pl.BlockSpec((BN, 1), lambda i: (i, 0)))
    in_specs += [w_spec, w_spec, b_spec]
    if b_has_x:
        in_specs.append(w_spec)
    in_specs += [w_spec, b_spec]
    if emit_out:
        in_specs.append(pl.BlockSpec((D, D), lambda i: (0, 0)))
        in_specs.append(pl.BlockSpec((1, D), lambda i: (0, 0)))
    out_specs = [pl.BlockSpec((NQ, BN, DQ), lambda i: (0, i, 0))] * 2
    out_shape = [jax.ShapeDtypeStruct((NQ, NPAD, DQ), jnp.float32)] * 2
    if emit_out:
        out_specs = out_specs + [pl.BlockSpec((BN, D), lambda i: (i, 0))]
        out_shape = out_shape + [jax.ShapeDtypeStruct((NPAD, D),
                                                      jnp.float32)]

    def body(*refs):
        it = iter(refs)
        pa = next(it)
        pb = next(it)
        h1 = next(it)
        iaB = next(it) if b_has_x else None
        h0 = next(it)
        invd = next(it)
        WxA = next(it)
        WhA = next(it)
        bA = next(it)
        WxB = next(it) if b_has_x else None
        WhB = next(it)
        bB = next(it)
        if emit_out:
            ow = next(it)
            ob = next(it)
        h1n_ref = next(it)
        h0n_ref = next(it)
        out_ref = next(it) if emit_out else None

        def full(q):
            return jnp.concatenate([q[k] for k in range(NQ)], axis=1)

        iv = invd[...]
        fpa = full(pa) * iv
        rows = (pl.program_id(0) * BN
                + lax.broadcasted_iota(jnp.int32, (BN, 1), 0))
        msk = rows < N

        GA = (bA[...]
              + jnp.dot(fpa, WxA[...], preferred_element_type=jnp.float32)
              + jnp.dot(full(pb) * iv, WhA[...],
                        preferred_element_type=jnp.float32))
        uA = jax.nn.sigmoid(GA[:, :D])
        cA = jnp.tanh(GA[:, D:])
        h1n = jnp.where(msk, uA * full(h1) + (1.0 - uA) * cA, 0.0)

        GB = bB[...] + jnp.dot(fpa, WhB[...],
                               preferred_element_type=jnp.float32)
        if b_has_x:
            GB = GB + jnp.dot(full(iaB) * iv, WxB[...],
                              preferred_element_type=jnp.float32)
        uB = jax.nn.sigmoid(GB[:, :D])
        cB = jnp.tanh(GB[:, D:])
        h0n = jnp.where(msk, uB * full(h0) + (1.0 - uB) * cB, 0.0)

        for k in range(NQ):
            h1n_ref[k] = h1n[:, k * DQ:(k + 1) * DQ]
            h0n_ref[k] = h0n[:, k * DQ:(k + 1) * DQ]
        if emit_out:
            out_ref[...] = (jnp.dot(h1n, ow[...],
                                    preferred_element_type=jnp.float32)
                            + ob[...])

    return pl.pallas_call(body, grid=(NBLK,), in_specs=in_specs,
                          out_specs=out_specs, out_shape=out_shape)


def kernel(inputs, teacher_states, enc_W, enc_b, dec_W, dec_b, out_W, out_b,
           edge_index, batch_cnt):
    f32 = jnp.float32
    src = edge_index[0]
    dst = edge_index[1]
    E = src.shape[0]
    pad_e = EPAD - E
    src3 = jnp.concatenate(
        [src, jnp.full((pad_e,), N, jnp.int32)]).reshape(NS, CPT, CHUNK)
    dst3 = jnp.concatenate(
        [dst, jnp.zeros((pad_e,), jnp.int32)]).reshape(NS, CPT, CHUNK)
    zrow = jnp.zeros((LANES, DQ), f32)

    xs = jnp.zeros((SEQ, NQ, NPAD, DQ), f32).at[:, :, :N, :].set(
        jnp.moveaxis(inputs.reshape(SEQ, N, NQ, DQ), 2, 1))
    xtabs = [xs[i].reshape(NQ * NPAD, DQ) for i in range(SEQ)]
    ones_tab = jnp.zeros((NQ, NPAD, DQ), f32).at[:, :N, :].set(1.0)
    ones_tab = ones_tab.reshape(NQ * NPAD, DQ)

    agg1 = _sc_agg(1)
    agg2 = _sc_agg(2)
    gate_x = _gate(True, False, False)
    gate_xh = _gate(True, True, False)
    gate_h = _gate(False, True, False)
    gate_xh_o = _gate(True, True, True)

    def agg(*tabs):
        return _sc_agg(len(tabs))(*tabs, src3, dst3, zrow)[0]

    degp = agg(ones_tab)[0]
    deg = degp[0, :, 0]
    invd = (1.0 / jnp.maximum(deg, 1.0))[:, None]
    IA = [agg(xtabs[i])[0] for i in range(SEQ)]

    def mk(Ws, bs):
        Wx = jnp.concatenate([Ws[2], Ws[4]], axis=1)
        Wh = jnp.concatenate([Ws[3], Ws[5]], axis=1)
        b = jnp.concatenate([bs[2] + bs[3], bs[4] + bs[5]])[None, :]
        return Wx, Wh, b

    encp = [mk(enc_W[j], enc_b[j]) for j in range(NL)]
    decp = [mk(dec_W[j], dec_b[j]) for j in range(NL)]
    ob = out_b[None, :]

    def flat(h4):
        return h4.reshape(NQ * NPAD, DQ)

    pair_enc = _gate2(True, False)
    pair_ed = _gate2(False, False)
    pair_dec = _gate2(False, True)

    h0 = gate_x(IA[0], invd, encp[0][0], encp[0][2])[0]
    ap_h0 = agg(flat(h0))[0]
    h1 = gate_x(ap_h0, invd, encp[1][0], encp[1][2])[0]
    h0 = gate_xh(IA[1], ap_h0, h0, invd,
                 encp[0][0], encp[0][1], encp[0][2])[0]
    for i in range(1, SEQ):
        pa, pb = agg(flat(h0), flat(h1))[:2]
        if i < SEQ - 1:
            h1, h0 = pair_enc(pa, pb, h1, IA[i + 1], h0, invd,
                              encp[1][0], encp[1][1], encp[1][2],
                              encp[0][0], encp[0][1], encp[0][2])
        else:
            h1, h0 = pair_ed(pa, pb, h1, h0, invd,
                             encp[1][0], encp[1][1], encp[1][2],
                             decp[0][1], decp[0][2])

    outs = []
    for i in range(SEQ):
        pa, pb = agg(flat(h0), flat(h1))[:2]
        if i < SEQ - 1:
            h1, h0, o = pair_dec(pa, pb, h1, h0, invd,
                                 decp[1][0], decp[1][1], decp[1][2],
                                 decp[0][1], decp[0][2], out_W, ob)
        else:
            h1, o = gate_xh_o(pa, pb, h1, invd,
                              decp[1][0], decp[1][1], decp[1][2], out_W, ob)
        outs.append(o[:N])
    return jnp.stack(outs)


# speedup vs baseline: 1.1866x; 1.0431x over previous
import functools

import jax
import jax.numpy as jnp
from jax import lax
from jax.experimental import pallas as pl
from jax.experimental.pallas import tpu as pltpu
from jax.experimental.pallas import tpu_sc as plsc

SEQ = 12
NL = 2
N = 10000
D = 128
D2 = 2 * D
NQ = 4
DQ = D // NQ
NPAD = 10240
CHUNK = 512
NBUF = 2
LAG = 1
NC = 2
NS = 16
RPT = NPAD // NS
CPT = 40
EPAD = NS * CPT * CHUNK
BN = 1280
NBLK = NPAD // BN
LANES = 128


@functools.lru_cache(maxsize=None)
def _sc_agg(n_tab: int):
    mesh = plsc.VectorSubcoreMesh(core_axis_name="c", subcore_axis_name="s")
    out_type = [jax.ShapeDtypeStruct((n_tab, NQ, NPAD, DQ), jnp.float32)]
    scratch_types = [
        pltpu.VMEM((CPT, CHUNK), jnp.int32),
        pltpu.VMEM((CPT, CHUNK), jnp.int32),
        pltpu.VMEM((NBUF, CHUNK, DQ), jnp.float32),
        pltpu.VMEM((LANES, DQ), jnp.float32),
        pltpu.VMEM_SHARED((NPAD, DQ), jnp.float32),
        pltpu.VMEM_SHARED((NPAD, DQ), jnp.float32),
    ] + [pltpu.SemaphoreType.DMA for _ in range(2 * NBUF)]

    def body(*refs):
        it = iter(refs)
        tabs = [next(it) for _ in range(n_tab)]
        src3 = next(it)
        dst3 = next(it)
        zrow_h = next(it)
        out = next(it)
        srcv = next(it)
        dstv = next(it)
        ring = next(it)
        zbuf = next(it)
        tstage = next(it)
        acc = next(it)
        gsems = [next(it) for _ in range(NBUF)]
        ssems = [next(it) for _ in range(NBUF)]

        cid = lax.axis_index("c")
        sid = lax.axis_index("s")
        r0 = sid * RPT

        pltpu.sync_copy(src3.at[sid], srcv)
        pltpu.sync_copy(dst3.at[sid], dstv)
        pltpu.sync_copy(zrow_h, zbuf)

        for t in range(n_tab):
            for qq in range(NQ // NC):
                qg = cid * (NQ // NC) + qq
                pltpu.sync_copy(tabs[t].at[pl.ds(qg * NPAD + r0, RPT)],
                                tstage.at[pl.ds(r0, RPT)])
                for k in range(RPT // LANES):
                    pltpu.sync_copy(zbuf,
                                    acc.at[pl.ds(r0 + k * LANES, LANES)])
                plsc.subcore_barrier()
                for b in range(NBUF):
                    pltpu.async_copy(tstage.at[srcv.at[b]], ring.at[b],
                                     gsems[b])

                def group(g, carry):
                    for b in range(NBUF):
                        c = g * NBUF + b
                        pltpu.make_async_copy(
                            tstage.at[srcv.at[c]], ring.at[b],
                            gsems[b]).wait()
                        pltpu.async_copy(ring.at[b], acc.at[dstv.at[c]],
                                         ssems[b], add=True)
                        br = (b - LAG) % NBUF
                        cr = c - LAG
                        nxt = cr + NBUF

                        @pl.when((cr >= 0) & (nxt < CPT))
                        def _():
                            pltpu.make_async_copy(
                                ring.at[br], acc.at[dstv.at[cr]],
                                ssems[br]).wait()
                            pltpu.async_copy(
                                tstage.at[srcv.at[nxt]], ring.at[br],
                                gsems[br])
                    return carry

                lax.fori_loop(0, CPT // NBUF, group, 0)
                for b in range(NBUF):
                    c = CPT - NBUF + b
                    pltpu.make_async_copy(
                        ring.at[b], acc.at[dstv.at[c]], ssems[b]).wait()
                plsc.subcore_barrier()
                pltpu.sync_copy(acc.at[pl.ds(r0, RPT)],
                                out.at[t, qg, pl.ds(r0, RPT)])

    return pl.kernel(body, out_type=out_type, mesh=mesh,
                     scratch_types=scratch_types,
                     compiler_params=pltpu.CompilerParams(
                         use_tc_tiling_on_sc=False))


@functools.lru_cache(maxsize=None)
def _gate(has_x: bool, has_h: bool, emit_out: bool):
    q_spec = pl.BlockSpec((NQ, BN, DQ), lambda i: (0, i, 0))
    in_specs = []
    if has_x:
        in_specs.append(q_spec)
    if has_h:
        in_specs.append(q_spec)
        in_specs.append(q_spec)
    in_specs.append(pl.BlockSpec((BN, 1), lambda i: (i, 0)))
    if has_x:
        in_specs.append(pl.BlockSpec((D, D2), lambda i: (0, 0)))
    if has_h:
        in_specs.append(pl.BlockSpec((D, D2), lambda i: (0, 0)))
    in_specs.append(pl.BlockSpec((1, D2), lambda i: (0, 0)))
    if emit_out:
        in_specs.append(pl.BlockSpec((D, D), lambda i: (0, 0)))
        in_specs.append(pl.BlockSpec((1, D), lambda i: (0, 0)))
    out_specs = [pl.BlockSpec((NQ, BN, DQ), lambda i: (0, i, 0))]
    out_shape = [jax.ShapeDtypeStruct((NQ, NPAD, DQ), jnp.float32)]
    if emit_out:
        out_specs.append(pl.BlockSpec((BN, D), lambda i: (i, 0)))
        out_shape.append(jax.ShapeDtypeStruct((NPAD, D), jnp.float32))

    def body(*refs):
        it = iter(refs)
        axp = next(it) if has_x else None
        if has_h:
            ahp = next(it)
            h4 = next(it)
        invd = next(it)
        Wx = next(it) if has_x else None
        Wh = next(it) if has_h else None
        bias = next(it)
        if emit_out:
            ow = next(it)
            ob = next(it)
        hn_ref = next(it)
        out_ref = next(it) if emit_out else None

        def full(q):
            return jnp.concatenate([q[k] for k in range(NQ)], axis=1)

        iv = invd[...]
        G = bias[...]
        if has_x:
            G = G + jnp.dot(full(axp) * iv, Wx[...],
                            preferred_element_type=jnp.float32)
        if has_h:
            G = G + jnp.dot(full(ahp) * iv, Wh[...],
                            preferred_element_type=jnp.float32)
        u = jax.nn.sigmoid(G[:, :D])
        c = jnp.tanh(G[:, D:])
        hn = u * full(h4) + (1.0 - u) * c if has_h else (1.0 - u) * c
        rows = (pl.program_id(0) * BN
                + lax.broadcasted_iota(jnp.int32, (BN, 1), 0))
        hn = jnp.where(rows < N, hn, 0.0)
        for k in range(NQ):
            hn_ref[k] = hn[:, k * DQ:(k + 1) * DQ]
        if emit_out:
            out_ref[...] = (jnp.dot(hn, ow[...],
                                    preferred_element_type=jnp.float32)
                            + ob[...])

    return pl.pallas_call(body, grid=(NBLK,), in_specs=in_specs,
                          out_specs=out_specs, out_shape=out_shape)


@functools.lru_cache(maxsize=None)
def _gate2(b_has_x: bool, emit_out: bool):
    q_spec = pl.BlockSpec((NQ, BN, DQ), lambda i: (0, i, 0))
    w_spec = pl.BlockSpec((D, D2), lambda i: (0, 0))
    b_spec = pl.BlockSpec((1, D2), lambda i: (0, 0))
    in_specs = [q_spec, q_spec, q_spec]
    if b_has_x:
        in_specs.append(q_spec)
    in_specs.append(q_spec)
    in_specs.append(pl.BlockSpec((BN, 1), lambda i: (i, 0)))
    in_specs += [w_spec, w_spec, b_spec]
    if b_has_x:
        in_specs.append(w_spec)
    in_specs += [w_spec, b_spec]
    if emit_out:
        in_specs.append(pl.BlockSpec((D, D), lambda i: (0, 0)))
        in_specs.append(pl.BlockSpec((1, D), lambda i: (0, 0)))
    out_specs = [pl.BlockSpec((NQ, BN, DQ), lambda i: (0, i, 0))] * 2
    out_shape = [jax.ShapeDtypeStruct((NQ, NPAD, DQ), jnp.float32)] * 2
    if emit_out:
        out_specs = out_specs + [pl.BlockSpec((BN, D), lambda i: (i, 0))]
        out_shape = out_shape + [jax.ShapeDtypeStruct((NPAD, D),
                                                      jnp.float32)]

    def body(*refs):
        it = iter(refs)
        pa = next(it)
        pb = next(it)
        h1 = next(it)
        iaB = next(it) if b_has_x else None
        h0 = next(it)
        invd = next(it)
        WxA = next(it)
        WhA = next(it)
        bA = next(it)
        WxB = next(it) if b_has_x else None
        WhB = next(it)
        bB = next(it)
        if emit_out:
            ow = next(it)
            ob = next(it)
        h1n_ref = next(it)
        h0n_ref = next(it)
        out_ref = next(it) if emit_out else None

        def full(q):
            return jnp.concatenate([q[k] for k in range(NQ)], axis=1)

        iv = invd[...]
        fpa = full(pa) * iv
        rows = (pl.program_id(0) * BN
                + lax.broadcasted_iota(jnp.int32, (BN, 1), 0))
        msk = rows < N

        GA = (bA[...]
              + jnp.dot(fpa, WxA[...], preferred_element_type=jnp.float32)
              + jnp.dot(full(pb) * iv, WhA[...],
                        preferred_element_type=jnp.float32))
        uA = jax.nn.sigmoid(GA[:, :D])
        cA = jnp.tanh(GA[:, D:])
        h1n = jnp.where(msk, uA * full(h1) + (1.0 - uA) * cA, 0.0)

        GB = bB[...] + jnp.dot(fpa, WhB[...],
                               preferred_element_type=jnp.float32)
        if b_has_x:
            GB = GB + jnp.dot(full(iaB) * iv, WxB[...],
                              preferred_element_type=jnp.float32)
        uB = jax.nn.sigmoid(GB[:, :D])
        cB = jnp.tanh(GB[:, D:])
        h0n = jnp.where(msk, uB * full(h0) + (1.0 - uB) * cB, 0.0)

        for k in range(NQ):
            h1n_ref[k] = h1n[:, k * DQ:(k + 1) * DQ]
            h0n_ref[k] = h0n[:, k * DQ:(k + 1) * DQ]
        if emit_out:
            out_ref[...] = (jnp.dot(h1n, ow[...],
                                    preferred_element_type=jnp.float32)
                            + ob[...])

    return pl.pallas_call(body, grid=(NBLK,), in_specs=in_specs,
                          out_specs=out_specs, out_shape=out_shape)


def kernel(inputs, teacher_states, enc_W, enc_b, dec_W, dec_b, out_W, out_b,
           edge_index, batch_cnt):
    f32 = jnp.float32
    src = edge_index[0]
    dst = edge_index[1]
    E = src.shape[0]
    pad_e = EPAD - E
    src3 = jnp.concatenate(
        [src, jnp.full((pad_e,), N, jnp.int32)]).reshape(NS, CPT, CHUNK)
    dst3 = jnp.concatenate(
        [dst, jnp.zeros((pad_e,), jnp.int32)]).reshape(NS, CPT, CHUNK)
    zrow = jnp.zeros((LANES, DQ), f32)

    xs = jnp.zeros((SEQ, NQ, NPAD, DQ), f32).at[:, :, :N, :].set(
        jnp.moveaxis(inputs.reshape(SEQ, N, NQ, DQ), 2, 1))
    xtabs = [xs[i].reshape(NQ * NPAD, DQ) for i in range(SEQ)]
    ones_tab = jnp.zeros((NQ, NPAD, DQ), f32).at[:, :N, :].set(1.0)
    ones_tab = ones_tab.reshape(NQ * NPAD, DQ)

    agg1 = _sc_agg(1)
    agg2 = _sc_agg(2)
    gate_x = _gate(True, False, False)
    gate_xh = _gate(True, True, False)
    gate_h = _gate(False, True, False)
    gate_xh_o = _gate(True, True, True)

    def agg(*tabs):
        return _sc_agg(len(tabs))(*tabs, src3, dst3, zrow)[0]

    degp = agg(ones_tab)[0]
    deg = degp[0, :, 0]
    invd = (1.0 / jnp.maximum(deg, 1.0))[:, None]
    IA = [agg(xtabs[i])[0] for i in range(SEQ)]

    def mk(Ws, bs):
        Wx = jnp.concatenate([Ws[2], Ws[4]], axis=1)
        Wh = jnp.concatenate([Ws[3], Ws[5]], axis=1)
        b = jnp.concatenate([bs[2] + bs[3], bs[4] + bs[5]])[None, :]
        return Wx, Wh, b

    encp = [mk(enc_W[j], enc_b[j]) for j in range(NL)]
    decp = [mk(dec_W[j], dec_b[j]) for j in range(NL)]
    ob = out_b[None, :]

    def flat(h4):
        return h4.reshape(NQ * NPAD, DQ)

    pair_enc = _gate2(True, False)
    pair_ed = _gate2(False, False)
    pair_dec = _gate2(False, True)

    h0 = gate_x(IA[0], invd, encp[0][0], encp[0][2])[0]
    ap_h0 = agg(flat(h0))[0]
    h1 = gate_x(ap_h0, invd, encp[1][0], encp[1][2])[0]
    h0 = gate_xh(IA[1], ap_h0, h0, invd,
                 encp[0][0], encp[0][1], encp[0][2])[0]
    for i in range(1, SEQ):
        pb = agg(flat(h1))[0]
        pa = agg(flat(h0))[0]
        if i < SEQ - 1:
            h1, h0 = pair_enc(pa, pb, h1, IA[i + 1], h0, invd,
                              encp[1][0], encp[1][1], encp[1][2],
                              encp[0][0], encp[0][1], encp[0][2])
        else:
            h1, h0 = pair_ed(pa, pb, h1, h0, invd,
                             encp[1][0], encp[1][1], encp[1][2],
                             decp[0][1], decp[0][2])

    outs = []
    for i in range(SEQ):
        pb = agg(flat(h1))[0]
        pa = agg(flat(h0))[0]
        if i < SEQ - 1:
            h1, h0, o = pair_dec(pa, pb, h1, h0, invd,
                                 decp[1][0], decp[1][1], decp[1][2],
                                 decp[0][1], decp[0][2], out_W, ob)
        else:
            h1, o = gate_xh_o(pa, pb, h1, invd,
                              decp[1][0], decp[1][1], decp[1][2], out_W, ob)
        outs.append(o[:N])
    return jnp.stack(outs)
